# baseline (speedup 1.0000x reference)
import functools

import jax
import jax.numpy as jnp
import numpy as np
from jax import lax
from jax.experimental import pallas as pl
from jax.experimental.pallas import tpu as pltpu
from jax.experimental.pallas import tpu_sc as plsc

B, N, C, K, H, NUM_CLASSES = 4, 2048, 64, 32, 8, 13
BLK1 = 512
BLKA = 128
TD = 128
INV_SQRT_HD = float(1.0 / np.sqrt(C // H))


def _ln(x, g, b, eps=1e-5):
    m = jnp.mean(x, axis=-1, keepdims=True)
    v = jnp.mean((x - m) ** 2, axis=-1, keepdims=True)
    return (x - m) / jnp.sqrt(v + eps) * g + b


def _k1_body(xyz_blk_ref, xyzT_ref, W_emb_ref, b_emb_ref, g1_ref, b1_ref,
             Wq_ref, bq_ref, bk_ref,
             feat_ref, tab_ref, qb_ref, gidx_ref):
    x = xyz_blk_ref[0]
    xaT = xyzT_ref[0]

    features = b_emb_ref[...][None, :] + (
        x[:, 0:1] * W_emb_ref[0:1, :]
        + x[:, 1:2] * W_emb_ref[1:2, :]
        + x[:, 2:3] * W_emb_ref[2:3, :])
    fn = _ln(features, g1_ref[...], b1_ref[...])
    qb = (jnp.dot(fn, Wq_ref[...], preferred_element_type=jnp.float32)
          + bq_ref[...] - bk_ref[...])

    feat_ref[0] = features
    tab_ref[0] = jnp.concatenate(
        [fn, x, jnp.zeros((BLK1, TD - C - 3), jnp.float32)], axis=1)
    qb_ref[0] = qb

    def _rne_bf16(v):
        ui = lax.bitcast_convert_type(v, jnp.uint32)
        r = (ui + 0x7FFF + ((ui >> 16) & 1)) & jnp.uint32(0xFFFF0000)
        return lax.bitcast_convert_type(r, jnp.float32)

    xxb = jnp.sum(x * x, axis=1, keepdims=True)
    xxa = jnp.sum(xaT * xaT, axis=0, keepdims=True)
    xr = _rne_bf16(x)
    xaTr = _rne_bf16(xaT)
    psum = (xr[:, 0:1] * xaTr[0:1, :]
            + xr[:, 1:2] * xaTr[1:2, :]
            + xr[:, 2:3] * xaTr[2:3, :])
    inner = -2.0 * psum
    X = -xxb - inner - xxa

    iota = lax.broadcasted_iota(jnp.int32, (BLK1, N), 1)
    cols = []
    for _ in range(K):
        m = jnp.max(X, axis=1, keepdims=True)
        am = jnp.min(jnp.where(X == m, iota, N), axis=1, keepdims=True)
        cols.append(am)
        X = jnp.where(iota == am, -jnp.inf, X)
    gidx_ref[0] = jnp.concatenate(cols, axis=1)


def _k3_body(G_ref, xyz_ref, qb_ref, feat_ref,
                  Wk_ref, Wv_ref, Wd1_ref, bd1_ref, Wd2_ref, bd2_ref,
                  gd_ref, bd_ref,
                  Wg1_ref, bg1_ref, gg_ref, bg_ref, Wg2_ref, bg2_ref,
                  bv_ref, Wo_ref, bo_ref, g2_ref, b2_ref, Ws_ref, bs_ref,
                  o_ref):
    R = BLKA * K
    G = G_ref[...]
    fnG = G[:, :C]
    xq = xyz_ref[0]
    qb = qb_ref[0]

    kk = jnp.dot(fnG, Wk_ref[...], preferred_element_type=jnp.float32)
    vv = jnp.dot(fnG, Wv_ref[...], preferred_element_type=jnp.float32)

    t1G = (G[:, C:C + 1] * Wd1_ref[0:1, :]
           + G[:, C + 1:C + 2] * Wd1_ref[1:2, :]
           + G[:, C + 2:C + 3] * Wd1_ref[2:3, :])
    t1q = (xq[:, 0:1] * Wd1_ref[0:1, :]
           + xq[:, 1:2] * Wd1_ref[1:2, :]
           + xq[:, 2:3] * Wd1_ref[2:3, :])

    d1 = (t1q[:, None, :] - t1G.reshape(BLKA, K, C)) + bd1_ref[...]
    h = jax.nn.relu(_ln(d1, gd_ref[...], bd_ref[...]))
    pe = (jnp.dot(h.reshape(R, C), Wd2_ref[...],
                  preferred_element_type=jnp.float32)
          + bd2_ref[...]).reshape(BLKA, K, C)

    attn_in = qb[:, None, :] - kk.reshape(BLKA, K, C) + pe
    a2 = (jnp.dot(attn_in.reshape(R, C), Wg1_ref[...],
                  preferred_element_type=jnp.float32) + bg1_ref[...])
    h2 = jax.nn.relu(_ln(a2, gg_ref[...], bg_ref[...]))
    scores = (jnp.dot(h2, Wg2_ref[...], preferred_element_type=jnp.float32)
              + bg2_ref[...]).reshape(BLKA, K, C)

    s = scores * INV_SQRT_HD
    smax = jnp.max(s, axis=1, keepdims=True)
    e = jnp.exp(s - smax)
    attn = e / jnp.sum(e, axis=1, keepdims=True)

    vpe = vv.reshape(BLKA, K, C) + pe + bv_ref[...]
    weighted = jnp.sum(attn * vpe, axis=1)

    out = (jnp.dot(weighted, Wo_ref[...], preferred_element_type=jnp.float32)
           + bo_ref[...])
    res = _ln(feat_ref[0] + out, g2_ref[...], b2_ref[...])
    o_ref[0] = (jnp.dot(res, Ws_ref[...], preferred_element_type=jnp.float32)
                + bs_ref[...])


_SC_NC, _SC_NS = 2, 16
_SC_NW = _SC_NC * _SC_NS
_ROWS = N * K
_PER_W = _ROWS // _SC_NW
_CH = 128
_ITERS = _PER_W // _CH


def _sc_gather_body(tab_hbm, idx_hbm, out_hbm,
                    idx0, idx1, rows0, rows1, sem0, sem1):
    wid = lax.axis_index("s") * _SC_NC + lax.axis_index("c")
    base = wid * _PER_W
    idx_v = (idx0, idx1)
    rows_v = (rows0, rows1)
    sems = (sem0, sem1)

    def start(j, p):
        pltpu.sync_copy(idx_hbm.at[pl.ds(base + j * _CH, _CH)], idx_v[p])
        pltpu.async_copy(tab_hbm.at[idx_v[p]], rows_v[p], sems[p])

    def finish(j, p):
        pltpu.make_async_copy(tab_hbm.at[idx_v[p]], rows_v[p], sems[p]).wait()
        pltpu.sync_copy(rows_v[p], out_hbm.at[pl.ds(base + j * _CH, _CH)])

    start(0, 0)

    def body(i, carry):
        for p in range(2):
            j = 2 * i + p
            jn = j + 1

            @pl.when(jn < _ITERS)
            def _():
                start(jn, 1 - p)

            finish(j, p)
        return carry

    lax.fori_loop(0, _ITERS // 2, body, 0)


def _sc_gather(tabf, idxf):
    f = functools.partial(
        pl.kernel,
        mesh=plsc.VectorSubcoreMesh(core_axis_name="c", subcore_axis_name="s"),
        out_type=jax.ShapeDtypeStruct((_ROWS, TD), jnp.float32),
        scratch_types=[
            pltpu.VMEM((_CH,), jnp.int32),
            pltpu.VMEM((_CH,), jnp.int32),
            pltpu.VMEM((_CH, TD), jnp.float32),
            pltpu.VMEM((_CH, TD), jnp.float32),
            pltpu.SemaphoreType.DMA,
            pltpu.SemaphoreType.DMA,
        ],
    )(_sc_gather_body)
    return f(tabf, idxf)


def _full(shape):
    nd = len(shape)
    return pl.BlockSpec(shape, lambda i_, _nd=nd: (0,) * _nd)


def kernel(xyz, W_emb, b_emb, Wq, bq, Wk, bk, Wv, bv, Wd1, bd1, gd, bd, Wd2,
           bd2, Wg1, bg1, gg, bg, Wg2, bg2, Wo, bo, g1, b1, g2, b2, Ws, bs):
    xyzT = jnp.swapaxes(xyz, 1, 2)

    k1 = pl.pallas_call(
        _k1_body,
        grid=(N // BLK1,),
        in_specs=[
            pl.BlockSpec((1, BLK1, 3), lambda i_: (0, i_, 0)),
            pl.BlockSpec((1, 3, N), lambda i_: (0, 0, 0)),
            _full((3, C)), _full((C,)), _full((C,)), _full((C,)),
            _full((C, C)), _full((C,)), _full((C,)),
        ],
        out_specs=[
            pl.BlockSpec((1, BLK1, C), lambda i_: (0, i_, 0)),
            pl.BlockSpec((1, BLK1, TD), lambda i_: (0, i_, 0)),
            pl.BlockSpec((1, BLK1, C), lambda i_: (0, i_, 0)),
            pl.BlockSpec((1, BLK1, K), lambda i_: (0, i_, 0)),
        ],
        out_shape=[
            jax.ShapeDtypeStruct((1, N, C), jnp.float32),
            jax.ShapeDtypeStruct((1, N, TD), jnp.float32),
            jax.ShapeDtypeStruct((1, N, C), jnp.float32),
            jax.ShapeDtypeStruct((1, N, K), jnp.int32),
        ],
    )

    per_b = [k1(xyz[b:b + 1], xyzT[b:b + 1], W_emb, b_emb, g1, b1, Wq, bq, bk)
             for b in range(B)]

    nblk = N // BLKA
    full1 = pl.BlockSpec((1, BLKA, 3), lambda i_: (0, i_, 0))
    fullc = pl.BlockSpec((1, BLKA, C), lambda i_: (0, i_, 0))

    def _full1(shape):
        nd = len(shape)
        return pl.BlockSpec(shape, lambda i_, _nd=nd: (0,) * _nd)

    k3 = pl.pallas_call(
        _k3_body,
        grid=(nblk,),
        in_specs=[
            pl.BlockSpec((BLKA * K, TD), lambda i_: (i_, 0)),
            full1, fullc, fullc,
            _full1((C, C)), _full1((C, C)), _full1((3, C)), _full1((C,)),
            _full1((C, C)), _full1((C,)), _full1((C,)), _full1((C,)),
            _full1((C, C)), _full1((C,)), _full1((C,)), _full1((C,)),
            _full1((C, C)), _full1((C,)),
            _full1((C,)), _full1((C, C)), _full1((C,)), _full1((C,)),
            _full1((C,)), _full1((C, NUM_CLASSES)), _full1((NUM_CLASSES,)),
        ],
        out_specs=pl.BlockSpec((1, BLKA, NUM_CLASSES), lambda i_: (0, i_, 0)),
        out_shape=jax.ShapeDtypeStruct((1, N, NUM_CLASSES), jnp.float32),
    )

    outs = []
    for b in range(B):
        feat_b, tab_b, qb_b, gidx_b = per_b[b]
        G_b = _sc_gather(tab_b.reshape(N, TD), gidx_b.reshape(N * K))
        outs.append(k3(
            G_b, xyz[b:b + 1], qb_b, feat_b,
            Wk, Wv, Wd1, bd1, Wd2, bd2, gd, bd,
            Wg1, bg1, gg, bg, Wg2, bg2,
            bv, Wo, bo, g2, b2, Ws, bs))
    return jnp.concatenate(outs, axis=0)


# speedup vs baseline: 1.0482x; 1.0482x over previous
import functools

import jax
import jax.numpy as jnp
import numpy as np
from jax import lax
from jax.experimental import pallas as pl
from jax.experimental.pallas import tpu as pltpu
from jax.experimental.pallas import tpu_sc as plsc

B, N, C, K, H, NUM_CLASSES = 4, 2048, 64, 32, 8, 13
BLK1 = 512
BLKA = 128
TD = 128
INV_SQRT_HD = float(1.0 / np.sqrt(C // H))


def _ln(x, g, b, eps=1e-5):
    m = jnp.mean(x, axis=-1, keepdims=True)
    v = jnp.mean((x - m) ** 2, axis=-1, keepdims=True)
    return (x - m) / jnp.sqrt(v + eps) * g + b


def _k1_body(xyz_blk_ref, xyzT_ref, W_emb_ref, b_emb_ref, g1_ref, b1_ref,
             Wq_ref, bq_ref, bk_ref,
             feat_ref, tab_ref, qb_ref, gidx_ref):
    x = xyz_blk_ref[0]
    xaT = xyzT_ref[0]

    features = b_emb_ref[...][None, :] + (
        x[:, 0:1] * W_emb_ref[0:1, :]
        + x[:, 1:2] * W_emb_ref[1:2, :]
        + x[:, 2:3] * W_emb_ref[2:3, :])
    fn = _ln(features, g1_ref[...], b1_ref[...])
    qb = (jnp.dot(fn, Wq_ref[...], preferred_element_type=jnp.float32)
          + bq_ref[...] - bk_ref[...])

    feat_ref[0] = features
    tab_ref[0] = jnp.concatenate(
        [fn, x, jnp.zeros((BLK1, TD - C - 3), jnp.float32)], axis=1)
    qb_ref[0] = qb

    def _rne_bf16(v):
        ui = lax.bitcast_convert_type(v, jnp.uint32)
        r = (ui + 0x7FFF + ((ui >> 16) & 1)) & jnp.uint32(0xFFFF0000)
        return lax.bitcast_convert_type(r, jnp.float32)

    xxb = jnp.sum(x * x, axis=1, keepdims=True)
    xxa = jnp.sum(xaT * xaT, axis=0, keepdims=True)
    xr = _rne_bf16(x)
    xaTr = _rne_bf16(xaT)
    psum = (xr[:, 0:1] * xaTr[0:1, :]
            + xr[:, 1:2] * xaTr[1:2, :]
            + xr[:, 2:3] * xaTr[2:3, :])
    inner = -2.0 * psum
    X = -xxb - inner - xxa

    iota = lax.broadcasted_iota(jnp.int32, (BLK1, N), 1)
    cols = []
    for _ in range(K):
        m = jnp.max(X, axis=1, keepdims=True)
        am = jnp.min(jnp.where(X == m, iota, N), axis=1, keepdims=True)
        cols.append(am)
        X = jnp.where(iota == am, -jnp.inf, X)
    gidx_ref[0] = jnp.concatenate(cols, axis=1)


def _k3_body(G_ref, xyz_ref, qb_ref, feat_ref,
                  Wk_ref, Wv_ref, Wd1_ref, bd1_ref, Wd2_ref, bd2_ref,
                  gd_ref, bd_ref,
                  Wg1_ref, bg1_ref, gg_ref, bg_ref, Wg2_ref, bg2_ref,
                  bv_ref, Wo_ref, bo_ref, g2_ref, b2_ref, Ws_ref, bs_ref,
                  o_ref):
    R = BLKA * K
    G = G_ref[...]
    fnG = G[:, :C]
    xq = xyz_ref[0]
    qb = qb_ref[0]

    kk = jnp.dot(fnG, Wk_ref[...], preferred_element_type=jnp.float32)
    vv = jnp.dot(fnG, Wv_ref[...], preferred_element_type=jnp.float32)

    t1G = (G[:, C:C + 1] * Wd1_ref[0:1, :]
           + G[:, C + 1:C + 2] * Wd1_ref[1:2, :]
           + G[:, C + 2:C + 3] * Wd1_ref[2:3, :])
    t1q = (xq[:, 0:1] * Wd1_ref[0:1, :]
           + xq[:, 1:2] * Wd1_ref[1:2, :]
           + xq[:, 2:3] * Wd1_ref[2:3, :])

    d1 = (t1q[:, None, :] - t1G.reshape(BLKA, K, C)) + bd1_ref[...]
    h = jax.nn.relu(_ln(d1, gd_ref[...], bd_ref[...]))
    pe = (jnp.dot(h.reshape(R, C), Wd2_ref[...],
                  preferred_element_type=jnp.float32)
          + bd2_ref[...]).reshape(BLKA, K, C)

    attn_in = qb[:, None, :] - kk.reshape(BLKA, K, C) + pe
    a2 = (jnp.dot(attn_in.reshape(R, C), Wg1_ref[...],
                  preferred_element_type=jnp.float32) + bg1_ref[...])
    h2 = jax.nn.relu(_ln(a2, gg_ref[...], bg_ref[...]))
    scores = (jnp.dot(h2, Wg2_ref[...], preferred_element_type=jnp.float32)
              + bg2_ref[...]).reshape(BLKA, K, C)

    s = scores * INV_SQRT_HD
    smax = jnp.max(s, axis=1, keepdims=True)
    e = jnp.exp(s - smax)
    attn = e / jnp.sum(e, axis=1, keepdims=True)

    vpe = vv.reshape(BLKA, K, C) + pe + bv_ref[...]
    weighted = jnp.sum(attn * vpe, axis=1)

    out = (jnp.dot(weighted, Wo_ref[...], preferred_element_type=jnp.float32)
           + bo_ref[...])
    res = _ln(feat_ref[0] + out, g2_ref[...], b2_ref[...])
    o_ref[0] = (jnp.dot(res, Ws_ref[...], preferred_element_type=jnp.float32)
                + bs_ref[...])


_SC_NC, _SC_NS = 2, 16
_SC_NW = _SC_NC * _SC_NS
_ROWS = N * K
_PER_W = _ROWS // _SC_NW
_CH = 128
_ITERS = _PER_W // _CH


def _sc_gather_body(tab_hbm, idx_hbm, out_hbm,
                    idx0, idx1, rows0, rows1, sem0, sem1):
    wid = lax.axis_index("s") * _SC_NC + lax.axis_index("c")
    base = wid * _PER_W
    idx_v = (idx0, idx1)
    rows_v = (rows0, rows1)
    sems = (sem0, sem1)

    def start(j, p):
        pltpu.sync_copy(idx_hbm.at[pl.ds(base + j * _CH, _CH)], idx_v[p])
        pltpu.async_copy(tab_hbm.at[idx_v[p]], rows_v[p], sems[p])

    def finish(j, p):
        pltpu.make_async_copy(tab_hbm.at[idx_v[p]], rows_v[p], sems[p]).wait()
        pltpu.sync_copy(rows_v[p], out_hbm.at[pl.ds(base + j * _CH, _CH)])

    start(0, 0)

    def body(i, carry):
        for p in range(2):
            j = 2 * i + p
            jn = j + 1

            @pl.when(jn < _ITERS)
            def _():
                start(jn, 1 - p)

            finish(j, p)
        return carry

    lax.fori_loop(0, _ITERS // 2, body, 0)


def _sc_gather(tabf, idxf):
    f = functools.partial(
        pl.kernel,
        mesh=plsc.VectorSubcoreMesh(core_axis_name="c", subcore_axis_name="s"),
        out_type=jax.ShapeDtypeStruct((_ROWS, TD), jnp.float32),
        scratch_types=[
            pltpu.VMEM((_CH,), jnp.int32),
            pltpu.VMEM((_CH,), jnp.int32),
            pltpu.VMEM((_CH, TD), jnp.float32),
            pltpu.VMEM((_CH, TD), jnp.float32),
            pltpu.SemaphoreType.DMA,
            pltpu.SemaphoreType.DMA,
        ],
    )(_sc_gather_body)
    return f(tabf, idxf)


def _full(shape):
    nd = len(shape)
    return pl.BlockSpec(shape, lambda b_, i_, _nd=nd: (0,) * _nd)


def kernel(xyz, W_emb, b_emb, Wq, bq, Wk, bk, Wv, bv, Wd1, bd1, gd, bd, Wd2,
           bd2, Wg1, bg1, gg, bg, Wg2, bg2, Wo, bo, g1, b1, g2, b2, Ws, bs):
    xyzT = jnp.swapaxes(xyz, 1, 2)

    feat, tab, qb, gidx = pl.pallas_call(
        _k1_body,
        grid=(B, N // BLK1),
        in_specs=[
            pl.BlockSpec((1, BLK1, 3), lambda b_, i_: (b_, i_, 0)),
            pl.BlockSpec((1, 3, N), lambda b_, i_: (b_, 0, 0)),
            _full((3, C)), _full((C,)), _full((C,)), _full((C,)),
            _full((C, C)), _full((C,)), _full((C,)),
        ],
        out_specs=[
            pl.BlockSpec((1, BLK1, C), lambda b_, i_: (b_, i_, 0)),
            pl.BlockSpec((1, BLK1, TD), lambda b_, i_: (b_, i_, 0)),
            pl.BlockSpec((1, BLK1, C), lambda b_, i_: (b_, i_, 0)),
            pl.BlockSpec((1, BLK1, K), lambda b_, i_: (b_, i_, 0)),
        ],
        out_shape=[
            jax.ShapeDtypeStruct((B, N, C), jnp.float32),
            jax.ShapeDtypeStruct((B, N, TD), jnp.float32),
            jax.ShapeDtypeStruct((B, N, C), jnp.float32),
            jax.ShapeDtypeStruct((B, N, K), jnp.int32),
        ],
    )(xyz, xyzT, W_emb, b_emb, g1, b1, Wq, bq, bk)

    nblk = N // BLKA
    full1 = pl.BlockSpec((1, BLKA, 3), lambda i_: (0, i_, 0))
    fullc = pl.BlockSpec((1, BLKA, C), lambda i_: (0, i_, 0))

    def _full1(shape):
        nd = len(shape)
        return pl.BlockSpec(shape, lambda i_, _nd=nd: (0,) * _nd)

    k3 = pl.pallas_call(
        _k3_body,
        grid=(nblk,),
        in_specs=[
            pl.BlockSpec((BLKA * K, TD), lambda i_: (i_, 0)),
            full1, fullc, fullc,
            _full1((C, C)), _full1((C, C)), _full1((3, C)), _full1((C,)),
            _full1((C, C)), _full1((C,)), _full1((C,)), _full1((C,)),
            _full1((C, C)), _full1((C,)), _full1((C,)), _full1((C,)),
            _full1((C, C)), _full1((C,)),
            _full1((C,)), _full1((C, C)), _full1((C,)), _full1((C,)),
            _full1((C,)), _full1((C, NUM_CLASSES)), _full1((NUM_CLASSES,)),
        ],
        out_specs=pl.BlockSpec((1, BLKA, NUM_CLASSES), lambda i_: (0, i_, 0)),
        out_shape=jax.ShapeDtypeStruct((1, N, NUM_CLASSES), jnp.float32),
    )

    outs = []
    for b in range(B):
        G_b = _sc_gather(tab[b], gidx[b].reshape(N * K))
        outs.append(k3(
            G_b, xyz[b:b + 1], qb[b:b + 1], feat[b:b + 1],
            Wk, Wv, Wd1, bd1, Wd2, bd2, gd, bd,
            Wg1, bg1, gg, bg, Wg2, bg2,
            bv, Wo, bo, g2, b2, Ws, bs))
    return jnp.concatenate(outs, axis=0)


# speedup vs baseline: 1.0536x; 1.0052x over previous
import functools

import jax
import jax.numpy as jnp
import numpy as np
from jax import lax
from jax.experimental import pallas as pl
from jax.experimental.pallas import tpu as pltpu
from jax.experimental.pallas import tpu_sc as plsc

B, N, C, K, H, NUM_CLASSES = 4, 2048, 64, 32, 8, 13
BLK1 = 512
BLKA = 256
TD = 128
INV_SQRT_HD = float(1.0 / np.sqrt(C // H))


def _ln(x, g, b, eps=1e-5):
    m = jnp.mean(x, axis=-1, keepdims=True)
    v = jnp.mean((x - m) ** 2, axis=-1, keepdims=True)
    return (x - m) / jnp.sqrt(v + eps) * g + b


def _k1_body(xyz_blk_ref, xyzT_ref, W_emb_ref, b_emb_ref, g1_ref, b1_ref,
             Wq_ref, bq_ref, bk_ref,
             feat_ref, tab_ref, qb_ref, gidx_ref):
    x = xyz_blk_ref[0]
    xaT = xyzT_ref[0]

    features = b_emb_ref[...][None, :] + (
        x[:, 0:1] * W_emb_ref[0:1, :]
        + x[:, 1:2] * W_emb_ref[1:2, :]
        + x[:, 2:3] * W_emb_ref[2:3, :])
    fn = _ln(features, g1_ref[...], b1_ref[...])
    qb = (jnp.dot(fn, Wq_ref[...], preferred_element_type=jnp.float32)
          + bq_ref[...] - bk_ref[...])

    feat_ref[0] = features
    tab_ref[0] = jnp.concatenate(
        [fn, x, jnp.zeros((BLK1, TD - C - 3), jnp.float32)], axis=1)
    qb_ref[0] = qb

    def _rne_bf16(v):
        ui = lax.bitcast_convert_type(v, jnp.uint32)
        r = (ui + 0x7FFF + ((ui >> 16) & 1)) & jnp.uint32(0xFFFF0000)
        return lax.bitcast_convert_type(r, jnp.float32)

    xxb = jnp.sum(x * x, axis=1, keepdims=True)
    xxa = jnp.sum(xaT * xaT, axis=0, keepdims=True)
    xr = _rne_bf16(x)
    xaTr = _rne_bf16(xaT)
    psum = (xr[:, 0:1] * xaTr[0:1, :]
            + xr[:, 1:2] * xaTr[1:2, :]
            + xr[:, 2:3] * xaTr[2:3, :])
    inner = -2.0 * psum
    X = -xxb - inner - xxa

    iota = lax.broadcasted_iota(jnp.int32, (BLK1, N), 1)
    cols = []
    for _ in range(K):
        m = jnp.max(X, axis=1, keepdims=True)
        am = jnp.min(jnp.where(X == m, iota, N), axis=1, keepdims=True)
        cols.append(am)
        X = jnp.where(iota == am, -jnp.inf, X)
    gidx_ref[0] = jnp.concatenate(cols, axis=1)


def _k3_body(G_ref, xyz_ref, qb_ref, feat_ref,
                  Wk_ref, Wv_ref, Wd1_ref, bd1_ref, Wd2_ref, bd2_ref,
                  gd_ref, bd_ref,
                  Wg1_ref, bg1_ref, gg_ref, bg_ref, Wg2_ref, bg2_ref,
                  bv_ref, Wo_ref, bo_ref, g2_ref, b2_ref, Ws_ref, bs_ref,
                  o_ref):
    R = BLKA * K
    G = G_ref[...]
    fnG = G[:, :C]
    xq = xyz_ref[0]
    qb = qb_ref[0]

    kk = jnp.dot(fnG, Wk_ref[...], preferred_element_type=jnp.float32)
    vv = jnp.dot(fnG, Wv_ref[...], preferred_element_type=jnp.float32)

    t1G = (G[:, C:C + 1] * Wd1_ref[0:1, :]
           + G[:, C + 1:C + 2] * Wd1_ref[1:2, :]
           + G[:, C + 2:C + 3] * Wd1_ref[2:3, :])
    t1q = (xq[:, 0:1] * Wd1_ref[0:1, :]
           + xq[:, 1:2] * Wd1_ref[1:2, :]
           + xq[:, 2:3] * Wd1_ref[2:3, :])

    d1 = (t1q[:, None, :] - t1G.reshape(BLKA, K, C)) + bd1_ref[...]
    h = jax.nn.relu(_ln(d1, gd_ref[...], bd_ref[...]))
    pe = (jnp.dot(h.reshape(R, C), Wd2_ref[...],
                  preferred_element_type=jnp.float32)
          + bd2_ref[...]).reshape(BLKA, K, C)

    attn_in = qb[:, None, :] - kk.reshape(BLKA, K, C) + pe
    a2 = (jnp.dot(attn_in.reshape(R, C), Wg1_ref[...],
                  preferred_element_type=jnp.float32) + bg1_ref[...])
    h2 = jax.nn.relu(_ln(a2, gg_ref[...], bg_ref[...]))
    scores = (jnp.dot(h2, Wg2_ref[...], preferred_element_type=jnp.float32)
              + bg2_ref[...]).reshape(BLKA, K, C)

    s = scores * INV_SQRT_HD
    smax = jnp.max(s, axis=1, keepdims=True)
    e = jnp.exp(s - smax)
    attn = e / jnp.sum(e, axis=1, keepdims=True)

    vpe = vv.reshape(BLKA, K, C) + pe + bv_ref[...]
    weighted = jnp.sum(attn * vpe, axis=1)

    out = (jnp.dot(weighted, Wo_ref[...], preferred_element_type=jnp.float32)
           + bo_ref[...])
    res = _ln(feat_ref[0] + out, g2_ref[...], b2_ref[...])
    o_ref[0] = (jnp.dot(res, Ws_ref[...], preferred_element_type=jnp.float32)
                + bs_ref[...])


_SC_NC, _SC_NS = 2, 16
_SC_NW = _SC_NC * _SC_NS
_ROWS = N * K
_PER_W = _ROWS // _SC_NW
_CH = 128
_ITERS = _PER_W // _CH


def _sc_gather_body(tab_hbm, idx_hbm, out_hbm,
                    idx0, idx1, rows0, rows1, sem0, sem1):
    wid = lax.axis_index("s") * _SC_NC + lax.axis_index("c")
    base = wid * _PER_W
    idx_v = (idx0, idx1)
    rows_v = (rows0, rows1)
    sems = (sem0, sem1)

    def start(j, p):
        pltpu.sync_copy(idx_hbm.at[pl.ds(base + j * _CH, _CH)], idx_v[p])
        pltpu.async_copy(tab_hbm.at[idx_v[p]], rows_v[p], sems[p])

    def finish(j, p):
        pltpu.make_async_copy(tab_hbm.at[idx_v[p]], rows_v[p], sems[p]).wait()
        pltpu.sync_copy(rows_v[p], out_hbm.at[pl.ds(base + j * _CH, _CH)])

    start(0, 0)

    def body(i, carry):
        for p in range(2):
            j = 2 * i + p
            jn = j + 1

            @pl.when(jn < _ITERS)
            def _():
                start(jn, 1 - p)

            finish(j, p)
        return carry

    lax.fori_loop(0, _ITERS // 2, body, 0)


def _sc_gather(tabf, idxf):
    f = functools.partial(
        pl.kernel,
        mesh=plsc.VectorSubcoreMesh(core_axis_name="c", subcore_axis_name="s"),
        out_type=jax.ShapeDtypeStruct((_ROWS, TD), jnp.float32),
        scratch_types=[
            pltpu.VMEM((_CH,), jnp.int32),
            pltpu.VMEM((_CH,), jnp.int32),
            pltpu.VMEM((_CH, TD), jnp.float32),
            pltpu.VMEM((_CH, TD), jnp.float32),
            pltpu.SemaphoreType.DMA,
            pltpu.SemaphoreType.DMA,
        ],
    )(_sc_gather_body)
    return f(tabf, idxf)


def _full(shape):
    nd = len(shape)
    return pl.BlockSpec(shape, lambda b_, i_, _nd=nd: (0,) * _nd)


def kernel(xyz, W_emb, b_emb, Wq, bq, Wk, bk, Wv, bv, Wd1, bd1, gd, bd, Wd2,
           bd2, Wg1, bg1, gg, bg, Wg2, bg2, Wo, bo, g1, b1, g2, b2, Ws, bs):
    xyzT = jnp.swapaxes(xyz, 1, 2)

    feat, tab, qb, gidx = pl.pallas_call(
        _k1_body,
        grid=(B, N // BLK1),
        in_specs=[
            pl.BlockSpec((1, BLK1, 3), lambda b_, i_: (b_, i_, 0)),
            pl.BlockSpec((1, 3, N), lambda b_, i_: (b_, 0, 0)),
            _full((3, C)), _full((C,)), _full((C,)), _full((C,)),
            _full((C, C)), _full((C,)), _full((C,)),
        ],
        out_specs=[
            pl.BlockSpec((1, BLK1, C), lambda b_, i_: (b_, i_, 0)),
            pl.BlockSpec((1, BLK1, TD), lambda b_, i_: (b_, i_, 0)),
            pl.BlockSpec((1, BLK1, C), lambda b_, i_: (b_, i_, 0)),
            pl.BlockSpec((1, BLK1, K), lambda b_, i_: (b_, i_, 0)),
        ],
        out_shape=[
            jax.ShapeDtypeStruct((B, N, C), jnp.float32),
            jax.ShapeDtypeStruct((B, N, TD), jnp.float32),
            jax.ShapeDtypeStruct((B, N, C), jnp.float32),
            jax.ShapeDtypeStruct((B, N, K), jnp.int32),
        ],
    )(xyz, xyzT, W_emb, b_emb, g1, b1, Wq, bq, bk)

    nblk = N // BLKA
    full1 = pl.BlockSpec((1, BLKA, 3), lambda i_: (0, i_, 0))
    fullc = pl.BlockSpec((1, BLKA, C), lambda i_: (0, i_, 0))

    def _full1(shape):
        nd = len(shape)
        return pl.BlockSpec(shape, lambda i_, _nd=nd: (0,) * _nd)

    k3 = pl.pallas_call(
        _k3_body,
        grid=(nblk,),
        in_specs=[
            pl.BlockSpec((BLKA * K, TD), lambda i_: (i_, 0)),
            full1, fullc, fullc,
            _full1((C, C)), _full1((C, C)), _full1((3, C)), _full1((C,)),
            _full1((C, C)), _full1((C,)), _full1((C,)), _full1((C,)),
            _full1((C, C)), _full1((C,)), _full1((C,)), _full1((C,)),
            _full1((C, C)), _full1((C,)),
            _full1((C,)), _full1((C, C)), _full1((C,)), _full1((C,)),
            _full1((C,)), _full1((C, NUM_CLASSES)), _full1((NUM_CLASSES,)),
        ],
        out_specs=pl.BlockSpec((1, BLKA, NUM_CLASSES), lambda i_: (0, i_, 0)),
        out_shape=jax.ShapeDtypeStruct((1, N, NUM_CLASSES), jnp.float32),
    )

    outs = []
    for b in range(B):
        G_b = _sc_gather(tab[b], gidx[b].reshape(N * K))
        outs.append(k3(
            G_b, xyz[b:b + 1], qb[b:b + 1], feat[b:b + 1],
            Wk, Wv, Wd1, bd1, Wd2, bd2, gd, bd,
            Wg1, bg1, gg, bg, Wg2, bg2,
            bv, Wo, bo, g2, b2, Ws, bs))
    return jnp.concatenate(outs, axis=0)
